# Initial kernel scaffold; baseline (speedup 1.0000x reference)
import jax
import jax.numpy as jnp
from jax.experimental import pallas as pl


def kernel(x):
    raise NotImplementedError("write your pallas kernel here")


# speedup vs baseline: 54.8485x; 54.8485x over previous
import functools

import jax
import jax.numpy as jnp
from jax import lax
from jax.experimental import pallas as pl
from jax.experimental.pallas import tpu as pltpu
from jax.experimental.pallas import tpu_sc as plsc

_BINS = 256
_NROWS = 768
_E = 512 * 512
_CHUNK = 32768
_NBUF = 3
_NC = 2
_NS = 16
_NW = _NC * _NS
_LANES = 16
_UNROLL = 8


def _build(nrows=_NROWS, e=_E, chunk=_CHUNK, interpret=False):
    nch = e // chunk
    rows_per_w = nrows // _NW
    vpc = chunk // _LANES

    def body(x_hbm, out_hbm, b0, b1, b2, hist, outv, red_mn, red_mx,
             s0, s1, s2):
        bufs = [b0, b1, b2]
        sems = [s0, s1, s2]
        wid = lax.axis_index("s") * _NC + lax.axis_index("c")

        lane_base = lax.iota(jnp.int32, _LANES) * _BINS
        ones_v = jnp.full((_LANES,), 1.0, jnp.float32)
        zeros_v = jnp.zeros((_LANES,), jnp.float32)
        inf_v = jnp.full((_LANES,), jnp.inf, jnp.float32)
        ninf_v = jnp.full((_LANES,), -jnp.inf, jnp.float32)
        def lane_reduce_bcast(v, scratch, op):
            scratch[pl.ds(0, _LANES)] = v
            scratch[pl.ds(_LANES, _LANES)] = v
            for k in (8, 4, 2, 1):
                a = scratch[pl.ds(0, _LANES)]
                b = scratch[pl.ds(k, _LANES)]
                m = op(a, b)
                scratch[pl.ds(0, _LANES)] = m
                scratch[pl.ds(_LANES, _LANES)] = m
            return scratch[pl.ds(0, _LANES)]

        def fetch(c, b, r):
            return pltpu.async_copy(
                x_hbm.at[r, pl.ds(c * chunk, chunk)], bufs[b], sems[b])

        def reduce_chunk(buf, carry):
            def rbody(k, carry):
                mnv, mxv = carry
                base = k * (_LANES * _UNROLL)
                for u in range(_UNROLL):
                    v = buf[pl.ds(base + u * _LANES, _LANES)]
                    mnv = jnp.minimum(mnv, v)
                    mxv = jnp.maximum(mxv, v)
                return (mnv, mxv)
            return lax.fori_loop(0, vpc // _UNROLL, rbody, carry)

        def bin_chunk(buf, mn2, d):
            def bbody(k, _):
                base = k * (_LANES * _UNROLL)
                for u in range(_UNROLL):
                    v = buf[pl.ds(base + u * _LANES, _LANES)]
                    scaled = (v - mn2) / d * 256.0
                    fl = jnp.clip(scaled, 0.0, 255.0)
                    idx = fl.astype(jnp.int32) + lane_base
                    plsc.addupdate_scatter(hist, [idx], ones_v)
                return 0
            lax.fori_loop(0, vpc // _UNROLL, bbody, 0)

        def row_body(i, _):
            r = wid * rows_per_w + i

            handles = [fetch(c, c, r) for c in range(_NBUF)]

            def zbody(j, _):
                hist[pl.ds(j * _LANES, _LANES)] = zeros_v
                return 0
            lax.fori_loop(0, _BINS, zbody, 0)

            carry = (inf_v, ninf_v)
            for c in range(nch):
                b = c % _NBUF
                handles[b].wait()
                carry = reduce_chunk(bufs[b], carry)
                if c + _NBUF < nch:
                    handles[b] = fetch(c + _NBUF, b, r)
            mnv, mxv = carry
            mn = lane_reduce_bcast(mnv, red_mn, jnp.minimum)
            mx = lane_reduce_bcast(mxv, red_mx, jnp.maximum)
            same = mn == mx
            mn2 = jnp.where(same, mn - 1.0, mn)
            mx2 = jnp.where(same, mx + 1.0, mx)
            d = mx2 - mn2

            sched = [(nch - _NBUF + k, (nch - _NBUF + k) % _NBUF, True)
                     for k in range(_NBUF)]
            for k in range(nch - _NBUF):
                sched.append((k, sched[k][1], False))
            for k, (c, b, resident) in enumerate(sched):
                if not resident:
                    handles[b].wait()
                bin_chunk(bufs[b], mn2, d)
                if k + _NBUF < nch:
                    nc_, nb, _unused = sched[k + _NBUF]
                    handles[nb] = fetch(nc_, nb, r)

            def fbody(j, _):
                acc = hist[pl.ds(j * _LANES, _LANES)]
                for lane in range(1, _LANES):
                    acc = acc + hist[pl.ds(lane * _BINS + j * _LANES,
                                           _LANES)]
                outv[pl.ds(j * _LANES, _LANES)] = acc
                return 0
            lax.fori_loop(0, _BINS // _LANES, fbody, 0)
            pltpu.sync_copy(outv, out_hbm.at[r])
            return 0

        lax.fori_loop(0, rows_per_w, row_body, 0)

    return pl.kernel(
        body,
        out_type=jax.ShapeDtypeStruct((nrows, _BINS), jnp.float32),
        mesh=plsc.VectorSubcoreMesh(core_axis_name="c",
                                    subcore_axis_name="s",
                                    num_cores=_NC,
                                    num_subcores=_NS),
        compiler_params=pltpu.CompilerParams(needs_layout_passes=False),
        scratch_types=[
            pltpu.VMEM((chunk,), jnp.float32),
            pltpu.VMEM((chunk,), jnp.float32),
            pltpu.VMEM((chunk,), jnp.float32),
            pltpu.VMEM((_LANES * _BINS,), jnp.float32),
            pltpu.VMEM((_BINS,), jnp.float32),
            pltpu.VMEM((2 * _LANES,), jnp.float32),
            pltpu.VMEM((2 * _LANES,), jnp.float32),
            pltpu.SemaphoreType.DMA,
            pltpu.SemaphoreType.DMA,
            pltpu.SemaphoreType.DMA,
        ],
        interpret=interpret,
    )


_hist_sc = _build()


@jax.jit
def kernel(x):
    b, c, h, w = x.shape
    rows = x.reshape(b * c, h * w)
    hist = _hist_sc(rows)
    return hist.reshape(b, c, _BINS)


# speedup vs baseline: 210.5333x; 3.8385x over previous
import functools

import jax
import jax.numpy as jnp
from jax import lax
from jax.experimental import pallas as pl
from jax.experimental.pallas import tpu as pltpu
from jax.experimental.pallas import tpu_sc as plsc

_BINS = 256
_NROWS = 768
_E = 512 * 512
_CHUNK = 32768
_NBUF = 3
_NC = 2
_NS = 16
_NW = _NC * _NS
_LANES = 16
_UNROLL = 8


def _build(nrows=_NROWS, e=_E, chunk=_CHUNK, interpret=False):
    nch = e // chunk
    rows_per_w = nrows // _NW
    vpc = chunk // _LANES

    def body(x_hbm, out_hbm, b0, b1, b2, hist, outv, red_mn, red_mx,
             s0, s1, s2):
        bufs = [b0, b1, b2]
        sems = [s0, s1, s2]
        wid = lax.axis_index("s") * _NC + lax.axis_index("c")

        lane_base = lax.iota(jnp.int32, _LANES) * _BINS
        ones_v = jnp.full((_LANES,), 1.0, jnp.float32)
        zeros_v = jnp.zeros((_LANES,), jnp.float32)
        inf_v = jnp.full((_LANES,), jnp.inf, jnp.float32)
        ninf_v = jnp.full((_LANES,), -jnp.inf, jnp.float32)
        def lane_reduce_bcast(v, scratch, op):
            scratch[pl.ds(0, _LANES)] = v
            scratch[pl.ds(_LANES, _LANES)] = v
            for k in (8, 4, 2, 1):
                a = scratch[pl.ds(0, _LANES)]
                b = scratch[pl.ds(k, _LANES)]
                m = op(a, b)
                scratch[pl.ds(0, _LANES)] = m
                scratch[pl.ds(_LANES, _LANES)] = m
            return scratch[pl.ds(0, _LANES)]

        def fetch(c, b, r):
            return pltpu.async_copy(
                x_hbm.at[r, pl.ds(c * chunk, chunk)], bufs[b], sems[b])

        def reduce_chunk(buf, carry):
            def rbody(k, c):
                mnv, mxv = c
                v = buf[pl.ds(k * _LANES, _LANES)]
                return (jnp.minimum(mnv, v), jnp.maximum(mxv, v))
            return plsc.parallel_loop(
                0, vpc, 1, unroll=_UNROLL, carry=carry)(rbody)

        def bin_chunk(buf, mn2, scale):
            def bbody(k):
                v = buf[pl.ds(k * _LANES, _LANES)]
                scaled = (v - mn2) * scale
                fl = jnp.clip(scaled, 0.0, 255.0)
                idx = fl.astype(jnp.int32) + lane_base
                plsc.addupdate_scatter(hist, [idx], ones_v)
            plsc.parallel_loop(0, vpc, 1, unroll=_UNROLL)(bbody)

        def row_body(i, _):
            r = wid * rows_per_w + i

            handles = [fetch(c, c, r) for c in range(_NBUF)]

            def zbody(j):
                hist[pl.ds(j * _LANES, _LANES)] = zeros_v
            plsc.parallel_loop(0, _BINS, 1, unroll=4)(zbody)

            carry = (inf_v, ninf_v)
            for c in range(nch):
                b = c % _NBUF
                handles[b].wait()
                carry = reduce_chunk(bufs[b], carry)
                if c + _NBUF < nch:
                    handles[b] = fetch(c + _NBUF, b, r)
            mnv, mxv = carry
            mn = lane_reduce_bcast(mnv, red_mn, jnp.minimum)
            mx = lane_reduce_bcast(mxv, red_mx, jnp.maximum)
            same = mn == mx
            mn2 = jnp.where(same, mn - 1.0, mn)
            mx2 = jnp.where(same, mx + 1.0, mx)
            scale = 256.0 / (mx2 - mn2)

            sched = [(nch - _NBUF + k, (nch - _NBUF + k) % _NBUF, True)
                     for k in range(_NBUF)]
            for k in range(nch - _NBUF):
                sched.append((k, sched[k][1], False))
            for k, (c, b, resident) in enumerate(sched):
                if not resident:
                    handles[b].wait()
                bin_chunk(bufs[b], mn2, scale)
                if k + _NBUF < nch:
                    nc_, nb, _unused = sched[k + _NBUF]
                    handles[nb] = fetch(nc_, nb, r)

            def fbody(j):
                acc = hist[pl.ds(j * _LANES, _LANES)]
                for lane in range(1, _LANES):
                    acc = acc + hist[pl.ds(lane * _BINS + j * _LANES,
                                           _LANES)]
                outv[pl.ds(j * _LANES, _LANES)] = acc
            plsc.parallel_loop(0, _BINS // _LANES, 1, unroll=2)(fbody)
            pltpu.sync_copy(outv, out_hbm.at[r])
            return 0

        lax.fori_loop(0, rows_per_w, row_body, 0)

    return pl.kernel(
        body,
        out_type=jax.ShapeDtypeStruct((nrows, _BINS), jnp.float32),
        mesh=plsc.VectorSubcoreMesh(core_axis_name="c",
                                    subcore_axis_name="s",
                                    num_cores=_NC,
                                    num_subcores=_NS),
        compiler_params=pltpu.CompilerParams(needs_layout_passes=False),
        scratch_types=[
            pltpu.VMEM((chunk,), jnp.float32),
            pltpu.VMEM((chunk,), jnp.float32),
            pltpu.VMEM((chunk,), jnp.float32),
            pltpu.VMEM((_LANES * _BINS,), jnp.float32),
            pltpu.VMEM((_BINS,), jnp.float32),
            pltpu.VMEM((2 * _LANES,), jnp.float32),
            pltpu.VMEM((2 * _LANES,), jnp.float32),
            pltpu.SemaphoreType.DMA,
            pltpu.SemaphoreType.DMA,
            pltpu.SemaphoreType.DMA,
        ],
        interpret=interpret,
    )


_hist_sc = _build()


@jax.jit
def kernel(x):
    b, c, h, w = x.shape
    rows = x.reshape(b * c, h * w)
    hist = _hist_sc(rows)
    return hist.reshape(b, c, _BINS)


# speedup vs baseline: 311.7323x; 1.4807x over previous
import functools

import jax
import jax.numpy as jnp
from jax import lax
from jax.experimental import pallas as pl
from jax.experimental.pallas import tpu as pltpu
from jax.experimental.pallas import tpu_sc as plsc

_BINS = 256
_NBUF = 3
_NC = 2
_NS = 16
_NW = _NC * _NS
_LANES = 16
_UNROLL = 8


def _build(nb=8, nci=96, h=512, w=512, chunk=32768):
    nrows = nb * nci
    e = h * w
    nch = e // chunk
    rows_per_w = nrows // _NW
    vpc = chunk // _LANES
    hrows = chunk // w
    spr = w // _LANES

    def body(x_hbm, out_hbm, b0, b1, b2, hist, outv, red_mn, red_mx,
             s0, s1, s2):
        bufs = [b0, b1, b2]
        sems = [s0, s1, s2]
        wid = lax.axis_index("s") * _NC + lax.axis_index("c")

        lane_base = lax.iota(jnp.int32, _LANES) * _BINS
        ones_v = jnp.full((_LANES,), 1.0, jnp.float32)
        zeros_v = jnp.zeros((_LANES,), jnp.float32)
        inf_v = jnp.full((_LANES,), jnp.inf, jnp.float32)
        ninf_v = jnp.full((_LANES,), -jnp.inf, jnp.float32)

        def lane_reduce_bcast(v, scratch, op):
            scratch[pl.ds(0, _LANES)] = v
            scratch[pl.ds(_LANES, _LANES)] = v
            for k in (8, 4, 2, 1):
                a = scratch[pl.ds(0, _LANES)]
                b = scratch[pl.ds(k, _LANES)]
                m = op(a, b)
                scratch[pl.ds(0, _LANES)] = m
                scratch[pl.ds(_LANES, _LANES)] = m
            return scratch[pl.ds(0, _LANES)]

        def fetch(c, b, bi, ci):
            return pltpu.async_copy(
                x_hbm.at[bi, ci, pl.ds(c * hrows, hrows), :],
                bufs[b], sems[b])

        def reduce_chunk(buf, carry):
            def rbody(k, c):
                mnv, mxv = c
                v = buf[k // spr, pl.ds((k % spr) * _LANES, _LANES)]
                return (jnp.minimum(mnv, v), jnp.maximum(mxv, v))
            return plsc.parallel_loop(
                0, vpc, 1, unroll=_UNROLL, carry=carry)(rbody)

        def bin_chunk(buf, mn2, scale):
            def bbody(k):
                v = buf[k // spr, pl.ds((k % spr) * _LANES, _LANES)]
                scaled = (v - mn2) * scale
                fl = jnp.clip(scaled, 0.0, 255.0)
                idx = fl.astype(jnp.int32) + lane_base
                plsc.addupdate_scatter(hist, [idx], ones_v)
            plsc.parallel_loop(0, vpc, 1, unroll=_UNROLL)(bbody)

        def row_body(i, _):
            r = wid * rows_per_w + i
            bi = r // nci
            ci = r % nci

            handles = [fetch(c, c, bi, ci) for c in range(_NBUF)]

            def zbody(j):
                hist[pl.ds(j * _LANES, _LANES)] = zeros_v
            plsc.parallel_loop(0, _BINS, 1, unroll=4)(zbody)

            carry = (inf_v, ninf_v)
            for c in range(nch):
                b = c % _NBUF
                handles[b].wait()
                carry = reduce_chunk(bufs[b], carry)
                if c + _NBUF < nch:
                    handles[b] = fetch(c + _NBUF, b, bi, ci)
            mnv, mxv = carry
            mn = lane_reduce_bcast(mnv, red_mn, jnp.minimum)
            mx = lane_reduce_bcast(mxv, red_mx, jnp.maximum)
            same = mn == mx
            mn2 = jnp.where(same, mn - 1.0, mn)
            mx2 = jnp.where(same, mx + 1.0, mx)
            scale = 256.0 / (mx2 - mn2)

            sched = [(nch - _NBUF + k, (nch - _NBUF + k) % _NBUF, True)
                     for k in range(_NBUF)]
            for k in range(nch - _NBUF):
                sched.append((k, sched[k][1], False))
            for k, (c, b, resident) in enumerate(sched):
                if not resident:
                    handles[b].wait()
                bin_chunk(bufs[b], mn2, scale)
                if k + _NBUF < nch:
                    nc_, nb_, _unused = sched[k + _NBUF]
                    handles[nb_] = fetch(nc_, nb_, bi, ci)

            def fbody(j):
                acc = hist[pl.ds(j * _LANES, _LANES)]
                for lane in range(1, _LANES):
                    acc = acc + hist[pl.ds(lane * _BINS + j * _LANES,
                                           _LANES)]
                outv[pl.ds(j * _LANES, _LANES)] = acc
            plsc.parallel_loop(0, _BINS // _LANES, 1, unroll=2)(fbody)
            pltpu.sync_copy(outv, out_hbm.at[r])
            return 0

        lax.fori_loop(0, rows_per_w, row_body, 0)

    return pl.kernel(
        body,
        out_type=jax.ShapeDtypeStruct((nrows, _BINS), jnp.float32),
        mesh=plsc.VectorSubcoreMesh(core_axis_name="c",
                                    subcore_axis_name="s",
                                    num_cores=_NC, num_subcores=_NS),
        compiler_params=pltpu.CompilerParams(needs_layout_passes=False,
                                             use_tc_tiling_on_sc=True),
        scratch_types=[
            pltpu.VMEM((hrows, w), jnp.float32),
            pltpu.VMEM((hrows, w), jnp.float32),
            pltpu.VMEM((hrows, w), jnp.float32),
            pltpu.VMEM((_LANES * _BINS,), jnp.float32),
            pltpu.VMEM((_BINS,), jnp.float32),
            pltpu.VMEM((2 * _LANES,), jnp.float32),
            pltpu.VMEM((2 * _LANES,), jnp.float32),
            pltpu.SemaphoreType.DMA,
            pltpu.SemaphoreType.DMA,
            pltpu.SemaphoreType.DMA,
        ],
    )


_hist_sc = _build()


@jax.jit
def kernel(x):
    b, c, h, w = x.shape
    hist = _hist_sc(x)
    return hist.reshape(b, c, _BINS)


# speedup vs baseline: 314.0555x; 1.0075x over previous
import functools

import jax
import jax.numpy as jnp
from jax import lax
from jax.experimental import pallas as pl
from jax.experimental.pallas import tpu as pltpu
from jax.experimental.pallas import tpu_sc as plsc

_BINS = 256
_NBUF = 3
_NC = 2
_NS = 16
_NW = _NC * _NS
_LANES = 16
_UNROLL = 8


def _build(nb=8, nci=96, h=512, w=512, chunk=32768):
    nrows = nb * nci
    e = h * w
    nch = e // chunk
    rows_per_w = nrows // _NW
    vpc = chunk // _LANES
    hrows = chunk // w
    spr = w // _LANES

    def body(x_hbm, out_hbm, b0, b1, b2, hist, outv, red_mn, red_mx,
             s0, s1, s2):
        bufs = [b0, b1, b2]
        sems = [s0, s1, s2]
        wid = lax.axis_index("s") * _NC + lax.axis_index("c")

        lane_base = lax.iota(jnp.int32, _LANES) * _BINS
        ones_v = jnp.full((_LANES,), 1.0, jnp.float32)
        zeros_v = jnp.zeros((_LANES,), jnp.float32)
        inf_v = jnp.full((_LANES,), jnp.inf, jnp.float32)
        ninf_v = jnp.full((_LANES,), -jnp.inf, jnp.float32)

        def lane_reduce_bcast(v, scratch, op):
            scratch[pl.ds(0, _LANES)] = v
            scratch[pl.ds(_LANES, _LANES)] = v
            for k in (8, 4, 2, 1):
                a = scratch[pl.ds(0, _LANES)]
                b = scratch[pl.ds(k, _LANES)]
                m = op(a, b)
                scratch[pl.ds(0, _LANES)] = m
                scratch[pl.ds(_LANES, _LANES)] = m
            return scratch[pl.ds(0, _LANES)]

        def fetch(c, b, bi, ci):
            return pltpu.async_copy(
                x_hbm.at[bi, ci, pl.ds(c * hrows, hrows), :],
                bufs[b], sems[b])

        def reduce_chunk(buf, carry):
            def rbody(k, c):
                mnv, mxv = c
                v = buf[k // spr, pl.ds((k % spr) * _LANES, _LANES)]
                return (jnp.minimum(mnv, v), jnp.maximum(mxv, v))
            return plsc.parallel_loop(
                0, vpc, 1, unroll=_UNROLL, carry=carry)(rbody)

        def bin_chunk(buf, mn2, scale):
            def bbody(k):
                v = buf[k // spr, pl.ds((k % spr) * _LANES, _LANES)]
                scaled = (v - mn2) * scale
                fl = jnp.minimum(scaled, 255.0)
                idx = fl.astype(jnp.int32) + lane_base
                plsc.addupdate_scatter(hist, [idx], ones_v)
            plsc.parallel_loop(0, vpc, 1, unroll=2 * _UNROLL)(bbody)

        def row_body(i, _):
            r = wid * rows_per_w + i
            bi = r // nci
            ci = r % nci

            handles = [fetch(c, c, bi, ci) for c in range(_NBUF)]

            def zbody(j):
                hist[pl.ds(j * _LANES, _LANES)] = zeros_v
            plsc.parallel_loop(0, _BINS, 1, unroll=4)(zbody)

            carry = (inf_v, ninf_v)
            for c in range(nch):
                b = c % _NBUF
                handles[b].wait()
                carry = reduce_chunk(bufs[b], carry)
                if c + _NBUF < nch:
                    handles[b] = fetch(c + _NBUF, b, bi, ci)
            mnv, mxv = carry
            mn = lane_reduce_bcast(mnv, red_mn, jnp.minimum)
            mx = lane_reduce_bcast(mxv, red_mx, jnp.maximum)
            same = mn == mx
            mn2 = jnp.where(same, mn - 1.0, mn)
            mx2 = jnp.where(same, mx + 1.0, mx)
            scale = 256.0 / (mx2 - mn2)

            sched = [(nch - _NBUF + k, (nch - _NBUF + k) % _NBUF, True)
                     for k in range(_NBUF)]
            for k in range(nch - _NBUF):
                sched.append((k, sched[k][1], False))
            for k, (c, b, resident) in enumerate(sched):
                if not resident:
                    handles[b].wait()
                bin_chunk(bufs[b], mn2, scale)
                if k + _NBUF < nch:
                    nc_, nb_, _unused = sched[k + _NBUF]
                    handles[nb_] = fetch(nc_, nb_, bi, ci)

            def fbody(j):
                acc = hist[pl.ds(j * _LANES, _LANES)]
                for lane in range(1, _LANES):
                    acc = acc + hist[pl.ds(lane * _BINS + j * _LANES,
                                           _LANES)]
                outv[pl.ds(j * _LANES, _LANES)] = acc
            plsc.parallel_loop(0, _BINS // _LANES, 1, unroll=2)(fbody)
            pltpu.sync_copy(outv, out_hbm.at[r])
            return 0

        lax.fori_loop(0, rows_per_w, row_body, 0)

    return pl.kernel(
        body,
        out_type=jax.ShapeDtypeStruct((nrows, _BINS), jnp.float32),
        mesh=plsc.VectorSubcoreMesh(core_axis_name="c",
                                    subcore_axis_name="s",
                                    num_cores=_NC, num_subcores=_NS),
        compiler_params=pltpu.CompilerParams(needs_layout_passes=False,
                                             use_tc_tiling_on_sc=True),
        scratch_types=[
            pltpu.VMEM((hrows, w), jnp.float32),
            pltpu.VMEM((hrows, w), jnp.float32),
            pltpu.VMEM((hrows, w), jnp.float32),
            pltpu.VMEM((_LANES * _BINS,), jnp.float32),
            pltpu.VMEM((_BINS,), jnp.float32),
            pltpu.VMEM((2 * _LANES,), jnp.float32),
            pltpu.VMEM((2 * _LANES,), jnp.float32),
            pltpu.SemaphoreType.DMA,
            pltpu.SemaphoreType.DMA,
            pltpu.SemaphoreType.DMA,
        ],
    )


_hist_sc = _build()


@jax.jit
def kernel(x):
    b, c, h, w = x.shape
    hist = _hist_sc(x)
    return hist.reshape(b, c, _BINS)


# speedup vs baseline: 382.8427x; 1.2190x over previous
import functools

import jax
import jax.numpy as jnp
from jax import lax
from jax.experimental import pallas as pl
from jax.experimental.pallas import tpu as pltpu
from jax.experimental.pallas import tpu_sc as plsc

_BINS = 256
_NBUF = 4
_NC = 2
_NS = 16
_NW = _NC * _NS
_LANES = 16
_UNROLL = 16
_NCI = 96
_H = 512
_W = 512
_SPLITS = 3
_RSPLIT = 8 * _NCI // _SPLITS
_RPW = _RSPLIT // _NW
_CHUNK = 16384
_NCH = (_H * _W) // _CHUNK
_HROWS = _CHUNK // _W
_SPR = _W // _LANES


def _tc_minmax(split):
    rg0 = split * (_RSPLIT // 8)

    def body(x_ref, mn_ref, mx_ref):
        v = x_ref[...]
        mn = jnp.min(v, axis=(0, 2, 3))
        mx = jnp.max(v, axis=(0, 2, 3))
        mn_ref[...] = jnp.broadcast_to(mn[:, None], (8, 128))
        mx_ref[...] = jnp.broadcast_to(mx[:, None], (8, 128))

    cpg = _NCI // 8
    return pl.pallas_call(
        body,
        grid=(_RSPLIT // 8,),
        in_specs=[pl.BlockSpec(
            (1, 8, _H, _W),
            lambda i: ((rg0 + i) // cpg, (rg0 + i) % cpg, 0, 0))],
        out_specs=[
            pl.BlockSpec((8, 128), lambda i: (i, 0)),
            pl.BlockSpec((8, 128), lambda i: (i, 0)),
        ],
        out_shape=[
            jax.ShapeDtypeStruct((_RSPLIT, 128), jnp.float32),
            jax.ShapeDtypeStruct((_RSPLIT, 128), jnp.float32),
        ],
    )


def _sc_bin(split):
    row0 = split * _RSPLIT

    def body(x_hbm, mn_hbm, mx_hbm, out_hbm, b0, b1, b2, b3,
             mnb, mxb, hist, outv, s0, s1, s2, s3, sm):
        bufs = [b0, b1, b2, b3]
        sems = [s0, s1, s2, s3]
        wid = lax.axis_index("s") * _NC + lax.axis_index("c")

        lane_base = lax.iota(jnp.int32, _LANES) * _BINS
        ones_v = jnp.full((_LANES,), 1.0, jnp.float32)
        zeros_v = jnp.zeros((_LANES,), jnp.float32)

        pltpu.async_copy(mn_hbm.at[pl.ds(wid * _RPW, _RPW), :], mnb, sm)
        pltpu.async_copy(mx_hbm.at[pl.ds(wid * _RPW, _RPW), :], mxb,
                         sm).wait()
        pltpu.make_async_copy(mn_hbm.at[pl.ds(0, _RPW), :], mnb, sm).wait()

        def fetch(r, c, b):
            bi = r // _NCI
            ci = r % _NCI
            return pltpu.async_copy(
                x_hbm.at[bi, ci, pl.ds(c * _HROWS, _HROWS), :],
                bufs[b], sems[b])

        def chunk_wait(b):
            pltpu.make_async_copy(
                x_hbm.at[0, 0, pl.ds(0, _HROWS), :], bufs[b],
                sems[b]).wait()

        def bin_chunk(buf, mn2, scale):
            def bbody(k):
                v = buf[k // _SPR, pl.ds((k % _SPR) * _LANES, _LANES)]
                scaled = (v - mn2) * scale
                fl = jnp.minimum(scaled, 255.0)
                idx = fl.astype(jnp.int32) + lane_base
                plsc.addupdate_scatter(hist, [idx], ones_v)
            plsc.parallel_loop(0, _CHUNK // _LANES, 1,
                               unroll=_UNROLL)(bbody)

        def row_body(i, _):
            r = row0 + wid * _RPW + i
            rn = row0 + wid * _RPW + jnp.minimum(i + 1, _RPW - 1)

            def zbody(j):
                hist[pl.ds(j * _LANES, _LANES)] = zeros_v
            plsc.parallel_loop(0, _BINS, 1, unroll=4)(zbody)

            mn = mnb[i, pl.ds(0, _LANES)]
            mx = mxb[i, pl.ds(0, _LANES)]
            same = mn == mx
            mn2 = jnp.where(same, mn - 1.0, mn)
            mx2 = jnp.where(same, mx + 1.0, mx)
            scale = 256.0 / (mx2 - mn2)

            for c in range(_NCH):
                b = c % _NBUF
                chunk_wait(b)
                bin_chunk(bufs[b], mn2, scale)
                f = c + _NBUF
                if f < _NCH:
                    fetch(r, f, b)
                else:
                    fetch(rn, f - _NCH, b)

            def fbody(j):
                acc = hist[pl.ds(j * _LANES, _LANES)]
                for lane in range(1, _LANES):
                    acc = acc + hist[pl.ds(lane * _BINS + j * _LANES,
                                           _LANES)]
                outv[pl.ds(j * _LANES, _LANES)] = acc
            plsc.parallel_loop(0, _BINS // _LANES, 1, unroll=2)(fbody)
            pltpu.sync_copy(outv, out_hbm.at[r - row0])
            return 0

        r0 = row0 + wid * _RPW
        for c in range(_NBUF):
            fetch(r0, c, c)
        lax.fori_loop(0, _RPW, row_body, 0)
        for b in range(_NBUF):
            chunk_wait(b)

    return pl.kernel(
        body,
        out_type=jax.ShapeDtypeStruct((_RSPLIT, _BINS), jnp.float32),
        mesh=plsc.VectorSubcoreMesh(core_axis_name="c",
                                    subcore_axis_name="s",
                                    num_cores=_NC, num_subcores=_NS),
        compiler_params=pltpu.CompilerParams(needs_layout_passes=False,
                                             use_tc_tiling_on_sc=True),
        scratch_types=[
            pltpu.VMEM((_HROWS, _W), jnp.float32),
            pltpu.VMEM((_HROWS, _W), jnp.float32),
            pltpu.VMEM((_HROWS, _W), jnp.float32),
            pltpu.VMEM((_HROWS, _W), jnp.float32),
            pltpu.VMEM((_RPW, 128), jnp.float32),
            pltpu.VMEM((_RPW, 128), jnp.float32),
            pltpu.VMEM((_LANES * _BINS,), jnp.float32),
            pltpu.VMEM((_BINS,), jnp.float32),
            pltpu.SemaphoreType.DMA,
            pltpu.SemaphoreType.DMA,
            pltpu.SemaphoreType.DMA,
            pltpu.SemaphoreType.DMA,
            pltpu.SemaphoreType.DMA,
        ],
    )


_tc_stages = [_tc_minmax(s) for s in range(_SPLITS)]
_sc_stages = [_sc_bin(s) for s in range(_SPLITS)]


@jax.jit
def kernel(x):
    b, c, h, w = x.shape
    parts = []
    for s in range(_SPLITS):
        mn, mx = _tc_stages[s](x)
        parts.append(_sc_stages[s](x, mn, mx))
    hist = jnp.concatenate(parts, axis=0)
    return hist.reshape(b, c, _BINS)


# speedup vs baseline: 423.4325x; 1.1060x over previous
import functools

import jax
import jax.numpy as jnp
from jax import lax
from jax.experimental import pallas as pl
from jax.experimental.pallas import tpu as pltpu
from jax.experimental.pallas import tpu_sc as plsc

_BINS = 256
_NBUF = 4
_NC = 2
_NS = 16
_NW = _NC * _NS
_LANES = 16
_UNROLL = 8
_NCI = 96
_H = 512
_W = 512
_HH = _H // 2
_SPLITS = 3
_RSPLIT = 8 * _NCI // _SPLITS
_RPW = _RSPLIT // _NW
_CHUNK = 16384
_NCH = (_HH * _W) // _CHUNK
_HROWS = _CHUNK // _W
_SPR = _W // _LANES


def _tc_idx(split):
    rg0 = split * (_RSPLIT // 8)
    cpg = _NCI // 8

    def body(x_ref, out_ref):
        v = x_ref[...]
        mn = jnp.min(v, axis=(2, 3), keepdims=True)
        mx = jnp.max(v, axis=(2, 3), keepdims=True)
        same = mn == mx
        mn2 = jnp.where(same, mn - 1.0, mn)
        mx2 = jnp.where(same, mx + 1.0, mx)
        scale = 256.0 / (mx2 - mn2)
        idx = jnp.minimum((v - mn2) * scale, 255.0).astype(jnp.int32)
        ev = idx[0, :, :_HH, :]
        od = idx[0, :, _HH:, :]
        out_ref[...] = ev | (od << 16)

    return pl.pallas_call(
        body,
        grid=(_RSPLIT // 8,),
        in_specs=[pl.BlockSpec(
            (1, 8, _H, _W),
            lambda i: ((rg0 + i) // cpg, (rg0 + i) % cpg, 0, 0))],
        out_specs=pl.BlockSpec((8, _HH, _W), lambda i: (i, 0, 0)),
        out_shape=jax.ShapeDtypeStruct((_RSPLIT, _HH, _W), jnp.int32),
    )


def _sc_scatter(split):
    def body(idx_hbm, out_hbm, b0, b1, b2, b3, hist, outb,
             s0, s1, s2, s3):
        bufs = [b0, b1, b2, b3]
        sems = [s0, s1, s2, s3]
        wid = lax.axis_index("s") * _NC + lax.axis_index("c")

        lane_base = lax.iota(jnp.int32, _LANES) * _BINS
        ones_v = jnp.full((_LANES,), 1.0, jnp.float32)
        zeros_v = jnp.zeros((_LANES,), jnp.float32)
        lo_mask = jnp.full((_LANES,), 0xFFFF, jnp.int32)

        def fetch(r, c, b):
            return pltpu.async_copy(
                idx_hbm.at[r, pl.ds(c * _HROWS, _HROWS), :],
                bufs[b], sems[b])

        def chunk_wait(b):
            pltpu.make_async_copy(
                idx_hbm.at[0, pl.ds(0, _HROWS), :], bufs[b],
                sems[b]).wait()

        def scatter_chunk(buf):
            def bbody(k):
                v = buf[k // _SPR, pl.ds((k % _SPR) * _LANES, _LANES)]
                lo = (v & lo_mask) + lane_base
                hi = lax.shift_right_logical(v, 16) + lane_base
                plsc.addupdate_scatter(hist, [lo], ones_v)
                plsc.addupdate_scatter(hist, [hi], ones_v)
            plsc.parallel_loop(0, _CHUNK // _LANES, 1,
                               unroll=_UNROLL)(bbody)

        def row_body(i, _):
            r = wid * _RPW + i
            rn = wid * _RPW + jnp.minimum(i + 1, _RPW - 1)

            def zbody(j):
                hist[pl.ds(j * _LANES, _LANES)] = zeros_v
            plsc.parallel_loop(0, _BINS, 1, unroll=4)(zbody)

            for c in range(_NCH):
                b = c % _NBUF
                chunk_wait(b)
                scatter_chunk(bufs[b])
                f = c + _NBUF
                if f < _NCH:
                    fetch(r, f, b)
                else:
                    fetch(rn, f - _NCH, b)

            def fbody(j):
                acc = hist[pl.ds(j * _LANES, _LANES)]
                for lane in range(1, _LANES):
                    acc = acc + hist[pl.ds(lane * _BINS + j * _LANES,
                                           _LANES)]
                outb[i, pl.ds(j * _LANES, _LANES)] = acc
            plsc.parallel_loop(0, _BINS // _LANES, 1, unroll=2)(fbody)
            return 0

        for c in range(_NBUF):
            fetch(wid * _RPW, c, c)
        lax.fori_loop(0, _RPW, row_body, 0)
        for b in range(_NBUF):
            chunk_wait(b)
        pltpu.sync_copy(outb, out_hbm.at[pl.ds(wid * _RPW, _RPW), :])

    return pl.kernel(
        body,
        out_type=jax.ShapeDtypeStruct((_RSPLIT, _BINS), jnp.float32),
        mesh=plsc.VectorSubcoreMesh(core_axis_name="c",
                                    subcore_axis_name="s",
                                    num_cores=_NC, num_subcores=_NS),
        compiler_params=pltpu.CompilerParams(needs_layout_passes=False,
                                             use_tc_tiling_on_sc=True),
        scratch_types=[
            pltpu.VMEM((_HROWS, _W), jnp.int32),
            pltpu.VMEM((_HROWS, _W), jnp.int32),
            pltpu.VMEM((_HROWS, _W), jnp.int32),
            pltpu.VMEM((_HROWS, _W), jnp.int32),
            pltpu.VMEM((_LANES * _BINS,), jnp.float32),
            pltpu.VMEM((_RPW, _BINS), jnp.float32),
            pltpu.SemaphoreType.DMA,
            pltpu.SemaphoreType.DMA,
            pltpu.SemaphoreType.DMA,
            pltpu.SemaphoreType.DMA,
        ],
    )


_tc_stages = [_tc_idx(s) for s in range(_SPLITS)]
_sc_stages = [_sc_scatter(s) for s in range(_SPLITS)]


@jax.jit
def kernel(x):
    b, c, h, w = x.shape
    parts = []
    for s in range(_SPLITS):
        idx = _tc_stages[s](x)
        parts.append(_sc_stages[s](idx))
    hist = jnp.concatenate(parts, axis=0)
    return hist.reshape(b, c, _BINS)
